# baseline (speedup 1.0000x reference)
import jax
import jax.numpy as jnp
from jax.experimental import pallas as pl
from jax.experimental.pallas import tpu as pltpu

_NCH = 8
_NSLOT = 12


def _row_softmax_kernel(x_hbm, o_hbm, buf, macc, sacc, isems, osems,
                        itsems, otsems):
    g = pl.program_id(0)
    p = pl.program_id(1)
    c = pl.program_id(2)
    ngroups = pl.num_programs(0)
    nj = ngroups * _NCH
    last = _NCH - 1

    n_cols = x_hbm.shape[1]
    step = (n_cols // _NCH) // 128 * 128
    tail = n_cols - _NCH * step
    csize = step + tail

    j = g * _NCH + c
    slot = jax.lax.rem(j, _NSLOT)

    def _gc(jj):
        if isinstance(jj, int):
            return jj // _NCH, jj % _NCH
        return jax.lax.div(jj, _NCH), jax.lax.rem(jj, _NCH)

    def in_main(jj, sl):
        gg, cc = _gc(jj)
        return pltpu.make_async_copy(
            x_hbm.at[pl.ds(gg * 8, 8), pl.ds(cc * step, step)],
            buf.at[sl, :, pl.ds(0, step)],
            isems.at[sl],
        )

    def in_tail(jj, sl):
        gg, _ = _gc(jj)
        return pltpu.make_async_copy(
            x_hbm.at[pl.ds(gg * 8, 8), pl.ds(_NCH * step, tail)],
            buf.at[sl, :, pl.ds(step, tail)],
            itsems.at[sl],
        )

    def out_main(jj, sl):
        gg, cc = _gc(jj)
        return pltpu.make_async_copy(
            buf.at[sl, :, pl.ds(0, step)],
            o_hbm.at[pl.ds(gg * 8, 8), pl.ds(cc * step, step)],
            osems.at[sl],
        )

    def out_tail(jj, sl):
        gg, _ = _gc(jj)
        return pltpu.make_async_copy(
            buf.at[sl, :, pl.ds(step, tail)],
            o_hbm.at[pl.ds(gg * 8, 8), pl.ds(_NCH * step, tail)],
            otsems.at[sl],
        )

    def start_in(jj, sl, is_last):
        in_main(jj, sl).start()

        @pl.when(is_last)
        def _():
            in_tail(jj, sl).start()

    @pl.when(jnp.logical_and(g == 0, jnp.logical_and(p == 0, c == 0)))
    def _():
        for jj in range(_NSLOT):
            start_in(jj, jj % _NSLOT, jj % _NCH == last)

    @pl.when(p == 0)
    def _():
        in_main(j, slot).wait()

        @pl.when(c == last)
        def _():
            in_tail(j, slot).wait()

        x = buf[slot]
        mc = jnp.max(x[:, :step], axis=1, keepdims=True)
        mt = jnp.max(x[:, step:], axis=1, keepdims=True)
        mc = jnp.maximum(mc, jnp.where(c == last, mt, -jnp.inf))
        macc[...] = jnp.where(c == 0, mc, jnp.maximum(macc[...], mc))

    @pl.when(p == 1)
    def _():
        x = buf[slot]
        e = jnp.exp(x - macc[...])
        buf[slot] = e
        sc_main = jnp.sum(e[:, :step], axis=1, keepdims=True)
        sc_tail = jnp.sum(e[:, step:], axis=1, keepdims=True)
        sc = sc_main + jnp.where(c == last, sc_tail, 0.0)
        sacc[...] = jnp.where(c == 0, sc, sacc[...] + sc)

    @pl.when(p == 2)
    def _():
        buf[slot] = buf[slot] * (1.0 / sacc[...])
        out_main(j, slot).start()

        @pl.when(c == last)
        def _():
            out_tail(j, slot).start()

        @pl.when(j >= 4)
        def _():
            jprev = j - 4
            slprev = jax.lax.rem(jprev, _NSLOT)
            _, ccprev = _gc(jprev)
            out_main(jprev, slprev).wait()

            @pl.when(ccprev == last)
            def _():
                out_tail(jprev, slprev).wait()

            @pl.when(j + 8 < nj)
            def _():
                _, ccnext = _gc(j + 8)
                start_in(j + 8, slprev, ccnext == last)

        @pl.when(j == nj - 1)
        def _():
            for dj in range(nj - 4, nj):
                out_main(dj, dj % _NSLOT).wait()
                if dj % _NCH == last:
                    out_tail(dj, dj % _NSLOT).wait()


def kernel(logits):
    n_rows, n_cols = logits.shape
    ngroups = n_rows // 8
    step = (n_cols // _NCH) // 128 * 128
    csize = n_cols - (_NCH - 1) * step
    return pl.pallas_call(
        _row_softmax_kernel,
        grid=(ngroups, 3, _NCH),
        in_specs=[pl.BlockSpec(memory_space=pl.ANY)],
        out_specs=pl.BlockSpec(memory_space=pl.ANY),
        out_shape=jax.ShapeDtypeStruct((n_rows, n_cols), logits.dtype),
        scratch_shapes=[
            pltpu.VMEM((_NSLOT, 8, csize), jnp.float32),
            pltpu.VMEM((8, 1), jnp.float32),
            pltpu.VMEM((8, 1), jnp.float32),
            pltpu.SemaphoreType.DMA((_NSLOT,)),
            pltpu.SemaphoreType.DMA((_NSLOT,)),
            pltpu.SemaphoreType.DMA((_NSLOT,)),
            pltpu.SemaphoreType.DMA((_NSLOT,)),
        ],
    )(logits)


# speedup vs baseline: 1.1015x; 1.1015x over previous
import jax
import jax.numpy as jnp
from jax.experimental import pallas as pl
from jax.experimental.pallas import tpu as pltpu

_NCH = 8
_NSLOT = 13


def _row_softmax_kernel(x_hbm, o_hbm, buf, macc, sacc, isems, osems,
                        itsems, otsems):
    g = pl.program_id(0)
    p = pl.program_id(1)
    c = pl.program_id(2)
    ngroups = pl.num_programs(0)
    nj = ngroups * _NCH
    last = _NCH - 1

    n_cols = x_hbm.shape[1]
    step = (n_cols // _NCH) // 128 * 128
    tail = n_cols - _NCH * step
    csize = step + tail

    j = g * _NCH + c
    slot = jax.lax.rem(j, _NSLOT)

    def _gc(jj):
        if isinstance(jj, int):
            return jj // _NCH, jj % _NCH
        return jax.lax.div(jj, _NCH), jax.lax.rem(jj, _NCH)

    def in_main(jj, sl):
        gg, cc = _gc(jj)
        return pltpu.make_async_copy(
            x_hbm.at[pl.ds(gg * 8, 8), pl.ds(cc * step, step)],
            buf.at[sl, :, pl.ds(0, step)],
            isems.at[sl],
        )

    def in_tail(jj, sl):
        gg, _ = _gc(jj)
        return pltpu.make_async_copy(
            x_hbm.at[pl.ds(gg * 8, 8), pl.ds(_NCH * step, tail)],
            buf.at[sl, :, pl.ds(step, tail)],
            itsems.at[sl],
        )

    def out_main(jj, sl):
        gg, cc = _gc(jj)
        return pltpu.make_async_copy(
            buf.at[sl, :, pl.ds(0, step)],
            o_hbm.at[pl.ds(gg * 8, 8), pl.ds(cc * step, step)],
            osems.at[sl],
        )

    def out_tail(jj, sl):
        gg, _ = _gc(jj)
        return pltpu.make_async_copy(
            buf.at[sl, :, pl.ds(step, tail)],
            o_hbm.at[pl.ds(gg * 8, 8), pl.ds(_NCH * step, tail)],
            otsems.at[sl],
        )

    def start_in(jj, sl, is_last):
        in_main(jj, sl).start()

        @pl.when(is_last)
        def _():
            in_tail(jj, sl).start()

    @pl.when(jnp.logical_and(g == 0, jnp.logical_and(p == 0, c == 0)))
    def _():
        for jj in range(_NSLOT):
            start_in(jj, jj % _NSLOT, jj % _NCH == last)

    @pl.when(p == 0)
    def _():
        in_main(j, slot).wait()

        @pl.when(c == last)
        def _():
            in_tail(j, slot).wait()

        x = buf[slot]
        mc = jnp.max(x[:, :step], axis=1, keepdims=True)
        mt = jnp.max(x[:, step:], axis=1, keepdims=True)
        mc = jnp.maximum(mc, jnp.where(c == last, mt, -jnp.inf))
        macc[...] = jnp.where(c == 0, mc, jnp.maximum(macc[...], mc))

    @pl.when(p == 1)
    def _():
        x = buf[slot]
        e = jnp.exp(x - macc[...])
        sc_main = jnp.sum(e[:, :step], axis=1, keepdims=True)
        sc_tail = jnp.sum(e[:, step:], axis=1, keepdims=True)
        sc = sc_main + jnp.where(c == last, sc_tail, 0.0)
        sacc[...] = jnp.where(c == 0, sc, sacc[...] + sc)

    @pl.when(p == 2)
    def _():
        x = buf[slot]
        buf[slot] = jnp.exp(x - macc[...]) * (1.0 / sacc[...])
        out_main(j, slot).start()

        @pl.when(c == last)
        def _():
            out_tail(j, slot).start()

        @pl.when(j >= 5)
        def _():
            jprev = j - 5
            slprev = jax.lax.rem(jprev, _NSLOT)
            _, ccprev = _gc(jprev)
            out_main(jprev, slprev).wait()

            @pl.when(ccprev == last)
            def _():
                out_tail(jprev, slprev).wait()

            @pl.when(j + 8 < nj)
            def _():
                _, ccnext = _gc(j + 8)
                start_in(j + 8, slprev, ccnext == last)

        @pl.when(j == nj - 1)
        def _():
            for dj in range(nj - 5, nj):
                out_main(dj, dj % _NSLOT).wait()
                if dj % _NCH == last:
                    out_tail(dj, dj % _NSLOT).wait()


def kernel(logits):
    n_rows, n_cols = logits.shape
    ngroups = n_rows // 8
    step = (n_cols // _NCH) // 128 * 128
    csize = n_cols - (_NCH - 1) * step
    return pl.pallas_call(
        _row_softmax_kernel,
        grid=(ngroups, 3, _NCH),
        in_specs=[pl.BlockSpec(memory_space=pl.ANY)],
        out_specs=pl.BlockSpec(memory_space=pl.ANY),
        out_shape=jax.ShapeDtypeStruct((n_rows, n_cols), logits.dtype),
        scratch_shapes=[
            pltpu.VMEM((_NSLOT, 8, csize), jnp.float32),
            pltpu.VMEM((8, 1), jnp.float32),
            pltpu.VMEM((8, 1), jnp.float32),
            pltpu.SemaphoreType.DMA((_NSLOT,)),
            pltpu.SemaphoreType.DMA((_NSLOT,)),
            pltpu.SemaphoreType.DMA((_NSLOT,)),
            pltpu.SemaphoreType.DMA((_NSLOT,)),
        ],
    )(logits)
